# Initial kernel scaffold; baseline (speedup 1.0000x reference)
import jax
import jax.numpy as jnp
from jax.experimental import pallas as pl


def kernel(query_embeddings, document_embeddings, query_pad_oov_mask, document_pad_oov_mask, pos_q, ln_g, ln_b, in_proj_w, in_proj_b, out_w, out_b, ff1_w, ff1_b, ff2_w, ff2_b, n1_g, n1_b, n2_g, n2_b, dense_w, pos_bias, pos_bias_abs):
    raise NotImplementedError("write your pallas kernel here")


# speedup vs baseline: 1.7601x; 1.7601x over previous
import functools
import math

import jax
import jax.numpy as jnp
from jax import lax
from jax.experimental import pallas as pl
from jax.experimental.pallas import tpu as pltpu

E = 512
H = 8
HD = E // H
FF = 2048
KK = 11
_MU = (1.0, 0.9, 0.7, 0.5, 0.3, 0.1, -0.1, -0.3, -0.5, -0.7, -0.9)
_SIG = (0.001,) + (0.1,) * 10
_SCALE = 1.0 / math.sqrt(HD)
_NEG = -1e9
_VMEM = pltpu.CompilerParams(
    dimension_semantics=("parallel",),
    vmem_limit_bytes=52 * 1024 * 1024,
)


def _ln2d(x, g, b):
    m = jnp.mean(x, axis=-1, keepdims=True)
    v = jnp.mean((x - m) ** 2, axis=-1, keepdims=True)
    return (x - m) / jnp.sqrt(v + 1e-5) * g + b


def _enc_layer_body(x_ref, kb_ref, pos_ref, lng_ref, lnb_ref,
                    iwT_ref, ib_ref, owT_ref, ob_ref,
                    f1T_ref, f1b_ref, f2T_ref, f2b_ref,
                    g1_ref, b1_ref, g2_ref, b2_ref,
                    o_ref, *, first, bs, s):
    ss = bs * s
    if first:
        x = (x_ref[...] + pos_ref[...][None, :, :]).reshape(ss, E)
        x = _ln2d(x, lng_ref[...], lnb_ref[...])
    else:
        x = x_ref[...].reshape(ss, E)
    kb = kb_ref[...].reshape(1, ss)
    if bs > 1:
        r = lax.broadcasted_iota(jnp.int32, (ss, ss), 0) // s
        c = lax.broadcasted_iota(jnp.int32, (ss, ss), 1) // s
        kb = kb + jnp.where(r == c, 0.0, 2.0 * _NEG)
    qkv = lax.dot_general(x, iwT_ref[...], (((1,), (0,)), ((), ()))) + ib_ref[...]
    ctxs = []
    for h in range(H):
        qh = qkv[:, h * HD:(h + 1) * HD] * _SCALE
        kh = qkv[:, E + h * HD:E + (h + 1) * HD]
        vh = qkv[:, 2 * E + h * HD:2 * E + (h + 1) * HD]
        sc = lax.dot_general(qh, kh, (((1,), (1,)), ((), ()))) + kb
        mx = jnp.max(sc, axis=-1, keepdims=True)
        ex = jnp.exp(sc - mx)
        a = ex / jnp.sum(ex, axis=-1, keepdims=True)
        ctxs.append(lax.dot_general(a, vh, (((1,), (0,)), ((), ()))))
    ctx = jnp.concatenate(ctxs, axis=-1)
    att = lax.dot_general(ctx, owT_ref[...], (((1,), (0,)), ((), ()))) + ob_ref[...]
    x1 = _ln2d(x + att, g1_ref[...], b1_ref[...])
    hff = jnp.maximum(
        lax.dot_general(x1, f1T_ref[...], (((1,), (0,)), ((), ()))) + f1b_ref[...], 0.0)
    ff = lax.dot_general(hff, f2T_ref[...], (((1,), (0,)), ((), ()))) + f2b_ref[...]
    x2 = _ln2d(x1 + ff, g2_ref[...], b2_ref[...])
    o_ref[...] = x2.reshape(bs, s, E)


def _enc_layer(x, kb, pos, lng, lnb, iwT, ib, owT, ob, f1T, f1b, f2T, f2b,
               g1, b1, g2, b2, *, first, bs):
    b, s, _ = x.shape
    full = lambda a: pl.BlockSpec(a.shape, lambda i: (0,) * a.ndim)
    body = functools.partial(_enc_layer_body, first=first, bs=bs, s=s)
    return pl.pallas_call(
        body,
        out_shape=jax.ShapeDtypeStruct((b, s, E), jnp.float32),
        grid=(b // bs,),
        in_specs=[
            pl.BlockSpec((bs, s, E), lambda i: (i, 0, 0)),
            pl.BlockSpec((1, 1, bs * s), lambda i: (i, 0, 0)),
            full(pos), full(lng), full(lnb),
            full(iwT), full(ib), full(owT), full(ob),
            full(f1T), full(f1b), full(f2T), full(f2b),
            full(g1), full(b1), full(g2), full(b2),
        ],
        out_specs=pl.BlockSpec((bs, s, E), lambda i: (i, 0, 0)),
        compiler_params=_VMEM,
        name=f"enc_layer_s{s}_{'first' if first else 'next'}",
    )(x, kb, pos, lng, lnb, iwT, ib, owT, ob, f1T, f1b, f2T, f2b, g1, b1, g2, b2)


def _pool_body(q_ref, d_ref, w_ref, cm_ref, o_ref, *, bs, nq, nd):
    q = q_ref[...].reshape(bs * nq, E)
    d = d_ref[...].reshape(bs * nd, E)
    qn = q / (jnp.sqrt(jnp.sum(q * q, axis=-1, keepdims=True)) + 1e-13)
    dn = d / (jnp.sqrt(jnp.sum(d * d, axis=-1, keepdims=True)) + 1e-13)
    cos = lax.dot_general(qn, dn, (((1,), (1,)), ((), ())))
    cm = cm_ref[...].reshape(1, bs * nd)
    if bs > 1:
        r = lax.broadcasted_iota(jnp.int32, (bs * nq, bs * nd), 0) // nq
        c = lax.broadcasted_iota(jnp.int32, (bs * nq, bs * nd), 1) // nd
        cmf = cm * (r == c).astype(jnp.float32)
    else:
        cmf = cm
    cols = []
    for k in range(KK):
        inv = 1.0 / (2.0 * _SIG[k] ** 2)
        ex = jnp.exp((cos - _MU[k]) ** 2 * (-inv)) * cmf
        cols.append(jnp.sum(ex, axis=-1, keepdims=True))
    pkq = jnp.concatenate(cols, axis=-1)
    lg = jnp.log(jnp.maximum(pkq, 1e-10))
    wq = w_ref[...].reshape(bs * nq, KK)
    contrib = (lg * wq).reshape(bs, nq, KK)
    s1 = jnp.sum(contrib, axis=2)
    s2 = jnp.sum(s1, axis=1, keepdims=True)
    o_ref[...] = s2.reshape(bs, 1, 1)


def _pool(q, d, w, cm, *, bs):
    b, nq, _ = q.shape
    nd = d.shape[1]
    body = functools.partial(_pool_body, bs=bs, nq=nq, nd=nd)
    return pl.pallas_call(
        body,
        out_shape=jax.ShapeDtypeStruct((b, 1, 1), jnp.float32),
        grid=(b // bs,),
        in_specs=[
            pl.BlockSpec((bs, nq, E), lambda i: (i, 0, 0)),
            pl.BlockSpec((bs, nd, E), lambda i: (i, 0, 0)),
            pl.BlockSpec((bs, nq, KK), lambda i: (i, 0, 0)),
            pl.BlockSpec((1, 1, bs * nd), lambda i: (i, 0, 0)),
        ],
        out_specs=pl.BlockSpec((bs, 1, 1), lambda i: (i, 0, 0)),
        compiler_params=_VMEM,
        name="rbf_pool",
    )(q, d, w, cm)


def kernel(query_embeddings, document_embeddings, query_pad_oov_mask,
           document_pad_oov_mask, pos_q, ln_g, ln_b, in_proj_w, in_proj_b,
           out_w, out_b, ff1_w, ff1_b, ff2_w, ff2_b, n1_g, n1_b, n2_g, n2_b,
           dense_w, pos_bias, pos_bias_abs):
    f32 = jnp.float32
    b, ql, _ = query_embeddings.shape
    dl = document_embeddings.shape[1]
    nlayers = in_proj_w.shape[0]
    bs_q = 8 if b % 8 == 0 else 1
    bs_p = 4 if b % 4 == 0 else 1

    qmask = query_pad_oov_mask.astype(f32)
    dmask = document_pad_oov_mask.astype(f32)

    qkb = jnp.where(qmask > 0, 0.0, _NEG).reshape(b // bs_q, 1, bs_q * ql)
    dkb = jnp.where(dmask > 0, 0.0, _NEG).reshape(b, 1, dl)

    nbins = pos_bias.shape[0] - 1
    bin_pct = 1.0 / nbins
    doc_len = dmask.sum(1)
    pos_range = (jnp.round(doc_len * bin_pct)[:, None]
                 * jnp.arange(nbins, dtype=f32)).astype(jnp.int32)
    marks = jnp.zeros_like(dmask).at[
        jnp.arange(b)[:, None], pos_range].set(1.0)
    pos_idx = (jnp.cumsum(marks, axis=1) * dmask).astype(jnp.int32)
    pbm = pos_bias[pos_idx]
    abs_factors = pos_bias_abs.shape[0] - 1
    abs_steps = math.ceil(pos_q.shape[1] / abs_factors)
    pos_selects = jnp.repeat(
        jnp.arange(1, abs_factors + 1), abs_steps).astype(f32)[:dl]
    abs_idx = (pos_selects[None, :] * dmask).astype(jnp.int32)
    abm = pos_bias_abs[abs_idx]
    cmul = (dmask * pbm * abm).reshape(b // bs_p, 1, bs_p * dl)
    wq = (qmask[:, :, None] * dense_w[0][None, None, :]).astype(f32)

    lng = ln_g.reshape(1, E)
    lnb = ln_b.reshape(1, E)
    pos_d = pos_q[0, :dl]
    pos_qq = pos_q[0, :ql]

    def run_encoder(x0, kb, pos, bs):
        x = x0
        for l in range(nlayers):
            x = _enc_layer(
                x, kb, pos, lng, lnb,
                in_proj_w[l].T, in_proj_b[l].reshape(1, 3 * E),
                out_w[l].T, out_b[l].reshape(1, E),
                ff1_w[l].T, ff1_b[l].reshape(1, FF),
                ff2_w[l].T, ff2_b[l].reshape(1, E),
                n1_g[l].reshape(1, E), n1_b[l].reshape(1, E),
                n2_g[l].reshape(1, E), n2_b[l].reshape(1, E),
                first=(l == 0), bs=bs)
        return x

    q_enc = run_encoder(query_embeddings.astype(f32), qkb, pos_qq, bs_q)
    d_enc = run_encoder(document_embeddings.astype(f32), dkb, pos_d, 1)

    score3 = _pool(q_enc, d_enc, wq, cmul, bs=bs_p)
    return score3[:, 0, 0]


# speedup vs baseline: 1.8595x; 1.0564x over previous
import functools
import math

import jax
import jax.numpy as jnp
from jax import lax
from jax.experimental import pallas as pl
from jax.experimental.pallas import tpu as pltpu

E = 512
H = 8
HD = E // H
FF = 2048
KK = 11
_MU = (1.0, 0.9, 0.7, 0.5, 0.3, 0.1, -0.1, -0.3, -0.5, -0.7, -0.9)
_SIG = (0.001,) + (0.1,) * 10
_SCALE = 1.0 / math.sqrt(HD)
_NEG = -1e9
_VMEM = pltpu.CompilerParams(
    dimension_semantics=("parallel",),
    vmem_limit_bytes=52 * 1024 * 1024,
)


def _ln2d(x, g, b):
    m = jnp.mean(x, axis=-1, keepdims=True)
    v = jnp.mean((x - m) ** 2, axis=-1, keepdims=True)
    return (x - m) / jnp.sqrt(v + 1e-5) * g + b


def _bf(v):
    return v.astype(jnp.bfloat16)


def _dot(a, b_, dims):
    return lax.dot_general(a, b_, (dims, ((), ())),
                           preferred_element_type=jnp.float32)


def _enc_layer_body(x_ref, kb_ref, pos_ref, lng_ref, lnb_ref,
                    iwT_ref, ib_ref, owT_ref, ob_ref,
                    f1T_ref, f1b_ref, f2T_ref, f2b_ref,
                    g1_ref, b1_ref, g2_ref, b2_ref,
                    o_ref, *, first, bs, s):
    ss = bs * s
    if first:
        x = (x_ref[...] + pos_ref[...][None, :, :]).reshape(ss, E)
        x = _ln2d(x, lng_ref[...], lnb_ref[...])
    else:
        x = x_ref[...].reshape(ss, E)
    kb = kb_ref[...].reshape(1, ss)
    if bs > 1:
        r = lax.broadcasted_iota(jnp.int32, (ss, ss), 0) // s
        c = lax.broadcasted_iota(jnp.int32, (ss, ss), 1) // s
        kb = kb + jnp.where(r == c, 0.0, 2.0 * _NEG)
    qkv = _dot(_bf(x), iwT_ref[...], ((1,), (0,))) + ib_ref[...]
    ctxs = []
    for h in range(H):
        qh = _bf(qkv[:, h * HD:(h + 1) * HD] * _SCALE)
        kh = _bf(qkv[:, E + h * HD:E + (h + 1) * HD])
        vh = _bf(qkv[:, 2 * E + h * HD:2 * E + (h + 1) * HD])
        sc = _dot(qh, kh, ((1,), (1,))) + kb
        mx = jnp.max(sc, axis=-1, keepdims=True)
        ex = jnp.exp(sc - mx)
        a = ex / jnp.sum(ex, axis=-1, keepdims=True)
        ctxs.append(_dot(_bf(a), vh, ((1,), (0,))))
    ctx = jnp.concatenate(ctxs, axis=-1)
    att = _dot(_bf(ctx), owT_ref[...], ((1,), (0,))) + ob_ref[...]
    x1 = _ln2d(x + att, g1_ref[...], b1_ref[...])
    hff = jnp.maximum(
        _dot(_bf(x1), f1T_ref[...], ((1,), (0,))) + f1b_ref[...], 0.0)
    ff = _dot(_bf(hff), f2T_ref[...], ((1,), (0,))) + f2b_ref[...]
    x2 = _ln2d(x1 + ff, g2_ref[...], b2_ref[...])
    o_ref[...] = x2.reshape(bs, s, E)


def _enc_layer(x, kb, pos, lng, lnb, iwT, ib, owT, ob, f1T, f1b, f2T, f2b,
               g1, b1, g2, b2, *, first, bs):
    b, s, _ = x.shape
    full = lambda a: pl.BlockSpec(a.shape, lambda i: (0,) * a.ndim)
    body = functools.partial(_enc_layer_body, first=first, bs=bs, s=s)
    return pl.pallas_call(
        body,
        out_shape=jax.ShapeDtypeStruct((b, s, E), jnp.float32),
        grid=(b // bs,),
        in_specs=[
            pl.BlockSpec((bs, s, E), lambda i: (i, 0, 0)),
            pl.BlockSpec((1, 1, bs * s), lambda i: (i, 0, 0)),
            full(pos), full(lng), full(lnb),
            full(iwT), full(ib), full(owT), full(ob),
            full(f1T), full(f1b), full(f2T), full(f2b),
            full(g1), full(b1), full(g2), full(b2),
        ],
        out_specs=pl.BlockSpec((bs, s, E), lambda i: (i, 0, 0)),
        compiler_params=_VMEM,
        name=f"enc_layer_s{s}_{'first' if first else 'next'}",
    )(x, kb, pos, lng, lnb, iwT, ib, owT, ob, f1T, f1b, f2T, f2b, g1, b1, g2, b2)


def _pool_body(q_ref, d_ref, w_ref, cm_ref, o_ref, *, bs, nq, nd):
    q = q_ref[...].reshape(bs * nq, E)
    d = d_ref[...].reshape(bs * nd, E)
    qn = q / (jnp.sqrt(jnp.sum(q * q, axis=-1, keepdims=True)) + 1e-13)
    dn = d / (jnp.sqrt(jnp.sum(d * d, axis=-1, keepdims=True)) + 1e-13)
    cos = _dot(_bf(qn), _bf(dn), ((1,), (1,)))
    cm = cm_ref[...].reshape(1, bs * nd)
    if bs > 1:
        r = lax.broadcasted_iota(jnp.int32, (bs * nq, bs * nd), 0) // nq
        c = lax.broadcasted_iota(jnp.int32, (bs * nq, bs * nd), 1) // nd
        cmf = cm * (r == c).astype(jnp.float32)
    else:
        cmf = cm
    cols = []
    for k in range(KK):
        inv = 1.0 / (2.0 * _SIG[k] ** 2)
        ex = jnp.exp((cos - _MU[k]) ** 2 * (-inv)) * cmf
        cols.append(jnp.sum(ex, axis=-1, keepdims=True))
    pkq = jnp.concatenate(cols, axis=-1)
    lg = jnp.log(jnp.maximum(pkq, 1e-10))
    wq = w_ref[...].reshape(bs * nq, KK)
    contrib = (lg * wq).reshape(bs, nq, KK)
    s1 = jnp.sum(contrib, axis=2)
    s2 = jnp.sum(s1, axis=1, keepdims=True)
    o_ref[...] = s2.reshape(bs, 1, 1)


def _pool(q, d, w, cm, *, bs):
    b, nq, _ = q.shape
    nd = d.shape[1]
    body = functools.partial(_pool_body, bs=bs, nq=nq, nd=nd)
    return pl.pallas_call(
        body,
        out_shape=jax.ShapeDtypeStruct((b, 1, 1), jnp.float32),
        grid=(b // bs,),
        in_specs=[
            pl.BlockSpec((bs, nq, E), lambda i: (i, 0, 0)),
            pl.BlockSpec((bs, nd, E), lambda i: (i, 0, 0)),
            pl.BlockSpec((bs, nq, KK), lambda i: (i, 0, 0)),
            pl.BlockSpec((1, 1, bs * nd), lambda i: (i, 0, 0)),
        ],
        out_specs=pl.BlockSpec((bs, 1, 1), lambda i: (i, 0, 0)),
        compiler_params=_VMEM,
        name="rbf_pool",
    )(q, d, w, cm)


def kernel(query_embeddings, document_embeddings, query_pad_oov_mask,
           document_pad_oov_mask, pos_q, ln_g, ln_b, in_proj_w, in_proj_b,
           out_w, out_b, ff1_w, ff1_b, ff2_w, ff2_b, n1_g, n1_b, n2_g, n2_b,
           dense_w, pos_bias, pos_bias_abs):
    f32 = jnp.float32
    b, ql, _ = query_embeddings.shape
    dl = document_embeddings.shape[1]
    nlayers = in_proj_w.shape[0]
    bs_q = 8 if b % 8 == 0 else 1
    bs_p = 4 if b % 4 == 0 else 1

    qmask = query_pad_oov_mask.astype(f32)
    dmask = document_pad_oov_mask.astype(f32)

    qkb = jnp.where(qmask > 0, 0.0, _NEG).reshape(b // bs_q, 1, bs_q * ql)
    dkb = jnp.where(dmask > 0, 0.0, _NEG).reshape(b, 1, dl)

    nbins = pos_bias.shape[0] - 1
    bin_pct = 1.0 / nbins
    doc_len = dmask.sum(1)
    pos_range = (jnp.round(doc_len * bin_pct)[:, None]
                 * jnp.arange(nbins, dtype=f32)).astype(jnp.int32)
    marks = jnp.zeros_like(dmask).at[
        jnp.arange(b)[:, None], pos_range].set(1.0)
    pos_idx = (jnp.cumsum(marks, axis=1) * dmask).astype(jnp.int32)
    pbm = pos_bias[pos_idx]
    abs_factors = pos_bias_abs.shape[0] - 1
    abs_steps = math.ceil(pos_q.shape[1] / abs_factors)
    pos_selects = jnp.repeat(
        jnp.arange(1, abs_factors + 1), abs_steps).astype(f32)[:dl]
    abs_idx = (pos_selects[None, :] * dmask).astype(jnp.int32)
    abm = pos_bias_abs[abs_idx]
    cmul = (dmask * pbm * abm).reshape(b // bs_p, 1, bs_p * dl)
    wq = (qmask[:, :, None] * dense_w[0][None, None, :]).astype(f32)

    lng = ln_g.reshape(1, E)
    lnb = ln_b.reshape(1, E)
    pos_d = pos_q[0, :dl]
    pos_qq = pos_q[0, :ql]

    def run_encoder(x0, kb, pos, bs):
        x = x0
        for l in range(nlayers):
            bf16 = jnp.bfloat16
            x = _enc_layer(
                x, kb, pos, lng, lnb,
                in_proj_w[l].T.astype(bf16), in_proj_b[l].reshape(1, 3 * E),
                out_w[l].T.astype(bf16), out_b[l].reshape(1, E),
                ff1_w[l].T.astype(bf16), ff1_b[l].reshape(1, FF),
                ff2_w[l].T.astype(bf16), ff2_b[l].reshape(1, E),
                n1_g[l].reshape(1, E), n1_b[l].reshape(1, E),
                n2_g[l].reshape(1, E), n2_b[l].reshape(1, E),
                first=(l == 0), bs=bs)
        return x

    q_enc = run_encoder(query_embeddings.astype(f32), qkb, pos_qq, bs_q)
    d_enc = run_encoder(document_embeddings.astype(f32), dkb, pos_d, 1)

    score3 = _pool(q_enc, d_enc, wq, cmul, bs=bs_p)
    return score3[:, 0, 0]


# speedup vs baseline: 1.8991x; 1.0213x over previous
import functools
import math

import jax
import jax.numpy as jnp
from jax import lax
from jax.experimental import pallas as pl
from jax.experimental.pallas import tpu as pltpu

E = 512
H = 8
HD = E // H
FF = 2048
KK = 11
_MU = (1.0, 0.9, 0.7, 0.5, 0.3, 0.1, -0.1, -0.3, -0.5, -0.7, -0.9)
_SIG = (0.001,) + (0.1,) * 10
_SCALE = 1.0 / math.sqrt(HD)
_NEG = -1e9
_VMEM = pltpu.CompilerParams(
    dimension_semantics=("parallel",),
    vmem_limit_bytes=52 * 1024 * 1024,
)


def _ln2d(x, g, b):
    m = jnp.mean(x, axis=-1, keepdims=True)
    v = jnp.mean((x - m) ** 2, axis=-1, keepdims=True)
    return (x - m) / jnp.sqrt(v + 1e-5) * g + b


def _bf(v):
    return v.astype(jnp.bfloat16)


def _dot(a, b_, dims):
    return lax.dot_general(a, b_, (dims, ((), ())),
                           preferred_element_type=jnp.float32)


def _enc_layer_body(x_ref, kb_ref, pos_ref, lng_ref, lnb_ref,
                    iwT_ref, ib_ref, owT_ref, ob_ref,
                    f1T_ref, f1b_ref, f2T_ref, f2b_ref,
                    g1_ref, b1_ref, g2_ref, b2_ref,
                    o_ref, *, first, bs, s):
    ss = bs * s
    if first:
        x = (x_ref[...] + pos_ref[...][None, :, :]).reshape(ss, E)
        x = _ln2d(x, lng_ref[...], lnb_ref[...])
    else:
        x = x_ref[...].reshape(ss, E)
    kb = kb_ref[...].reshape(1, ss)
    if bs > 1:
        r = lax.broadcasted_iota(jnp.int32, (ss, ss), 0) // s
        c = lax.broadcasted_iota(jnp.int32, (ss, ss), 1) // s
        kb = kb + jnp.where(r == c, 0.0, 2.0 * _NEG)
    qkv = _dot(_bf(x), iwT_ref[...], ((1,), (0,))) + ib_ref[...]
    ctxs = []
    for h in range(H):
        qh = _bf(qkv[:, h * HD:(h + 1) * HD] * _SCALE)
        kh = _bf(qkv[:, E + h * HD:E + (h + 1) * HD])
        vh = _bf(qkv[:, 2 * E + h * HD:2 * E + (h + 1) * HD])
        sc = _dot(qh, kh, ((1,), (1,))) + kb
        mx = jnp.max(sc, axis=-1, keepdims=True)
        ex = jnp.exp(sc - mx)
        a = ex / jnp.sum(ex, axis=-1, keepdims=True)
        ctxs.append(_dot(_bf(a), vh, ((1,), (0,))))
    ctx = jnp.concatenate(ctxs, axis=-1)
    att = _dot(_bf(ctx), owT_ref[...], ((1,), (0,))) + ob_ref[...]
    x1 = _ln2d(x + att, g1_ref[...], b1_ref[...])
    hff = jnp.maximum(
        _dot(_bf(x1), f1T_ref[...], ((1,), (0,))) + f1b_ref[...], 0.0)
    ff = _dot(_bf(hff), f2T_ref[...], ((1,), (0,))) + f2b_ref[...]
    x2 = _ln2d(x1 + ff, g2_ref[...], b2_ref[...])
    o_ref[...] = x2.reshape(bs, s, E)


def _enc_layer(x, kb, pos, lng, lnb, iwT, ib, owT, ob, f1T, f1b, f2T, f2b,
               g1, b1, g2, b2, *, first, bs):
    b, s, _ = x.shape
    full = lambda a: pl.BlockSpec(a.shape, lambda i: (0,) * a.ndim)
    body = functools.partial(_enc_layer_body, first=first, bs=bs, s=s)
    return pl.pallas_call(
        body,
        out_shape=jax.ShapeDtypeStruct((b, s, E), jnp.float32),
        grid=(b // bs,),
        in_specs=[
            pl.BlockSpec((bs, s, E), lambda i: (i, 0, 0)),
            pl.BlockSpec((1, 1, bs * s), lambda i: (i, 0, 0)),
            full(pos), full(lng), full(lnb),
            full(iwT), full(ib), full(owT), full(ob),
            full(f1T), full(f1b), full(f2T), full(f2b),
            full(g1), full(b1), full(g2), full(b2),
        ],
        out_specs=pl.BlockSpec((bs, s, E), lambda i: (i, 0, 0)),
        compiler_params=_VMEM,
        name=f"enc_layer_s{s}_{'first' if first else 'next'}",
    )(x, kb, pos, lng, lnb, iwT, ib, owT, ob, f1T, f1b, f2T, f2b, g1, b1, g2, b2)


def _pool_body(q_ref, d_ref, w_ref, cm_ref, o_ref, *, bs, nq, nd):
    q = q_ref[...].reshape(bs * nq, E)
    d = d_ref[...].reshape(bs * nd, E)
    qn = q / (jnp.sqrt(jnp.sum(q * q, axis=-1, keepdims=True)) + 1e-13)
    dn = d / (jnp.sqrt(jnp.sum(d * d, axis=-1, keepdims=True)) + 1e-13)
    cos = _dot(_bf(qn), _bf(dn), ((1,), (1,)))
    cm = cm_ref[...].reshape(1, bs * nd)
    wq = w_ref[...].reshape(bs * nq, KK)
    scores = []
    for i in range(bs):
        ci = cos[i * nq:(i + 1) * nq, i * nd:(i + 1) * nd]
        cmi = cm[:, i * nd:(i + 1) * nd]
        cols = []
        for k in range(KK):
            inv = 1.0 / (2.0 * _SIG[k] ** 2)
            ex = jnp.exp((ci - _MU[k]) ** 2 * (-inv)) * cmi
            cols.append(jnp.sum(ex, axis=-1, keepdims=True))
        pkq = jnp.concatenate(cols, axis=-1)
        lg = jnp.log(jnp.maximum(pkq, 1e-10))
        scores.append(jnp.sum(lg * wq[i * nq:(i + 1) * nq, :]))
    o_ref[...] = jnp.stack(scores).reshape(bs, 1, 1)


def _pool(q, d, w, cm, *, bs):
    b, nq, _ = q.shape
    nd = d.shape[1]
    body = functools.partial(_pool_body, bs=bs, nq=nq, nd=nd)
    return pl.pallas_call(
        body,
        out_shape=jax.ShapeDtypeStruct((b, 1, 1), jnp.float32),
        grid=(b // bs,),
        in_specs=[
            pl.BlockSpec((bs, nq, E), lambda i: (i, 0, 0)),
            pl.BlockSpec((bs, nd, E), lambda i: (i, 0, 0)),
            pl.BlockSpec((bs, nq, KK), lambda i: (i, 0, 0)),
            pl.BlockSpec((1, 1, bs * nd), lambda i: (i, 0, 0)),
        ],
        out_specs=pl.BlockSpec((bs, 1, 1), lambda i: (i, 0, 0)),
        compiler_params=_VMEM,
        name="rbf_pool",
    )(q, d, w, cm)


def kernel(query_embeddings, document_embeddings, query_pad_oov_mask,
           document_pad_oov_mask, pos_q, ln_g, ln_b, in_proj_w, in_proj_b,
           out_w, out_b, ff1_w, ff1_b, ff2_w, ff2_b, n1_g, n1_b, n2_g, n2_b,
           dense_w, pos_bias, pos_bias_abs):
    f32 = jnp.float32
    b, ql, _ = query_embeddings.shape
    dl = document_embeddings.shape[1]
    nlayers = in_proj_w.shape[0]
    bs_q = 8 if b % 8 == 0 else 1
    bs_p = 4 if b % 4 == 0 else 1

    qmask = query_pad_oov_mask.astype(f32)
    dmask = document_pad_oov_mask.astype(f32)

    qkb = jnp.where(qmask > 0, 0.0, _NEG).reshape(b // bs_q, 1, bs_q * ql)
    dkb = jnp.where(dmask > 0, 0.0, _NEG).reshape(b, 1, dl)

    nbins = pos_bias.shape[0] - 1
    bin_pct = 1.0 / nbins
    doc_len = dmask.sum(1)
    pos_range = (jnp.round(doc_len * bin_pct)[:, None]
                 * jnp.arange(nbins, dtype=f32)).astype(jnp.int32)
    marks = jnp.zeros_like(dmask).at[
        jnp.arange(b)[:, None], pos_range].set(1.0)
    pos_idx = (jnp.cumsum(marks, axis=1) * dmask).astype(jnp.int32)
    pbm = pos_bias[pos_idx]
    abs_factors = pos_bias_abs.shape[0] - 1
    abs_steps = math.ceil(pos_q.shape[1] / abs_factors)
    pos_selects = jnp.repeat(
        jnp.arange(1, abs_factors + 1), abs_steps).astype(f32)[:dl]
    abs_idx = (pos_selects[None, :] * dmask).astype(jnp.int32)
    abm = pos_bias_abs[abs_idx]
    cmul = (dmask * pbm * abm).reshape(b // bs_p, 1, bs_p * dl)
    wq = (qmask[:, :, None] * dense_w[0][None, None, :]).astype(f32)

    lng = ln_g.reshape(1, E)
    lnb = ln_b.reshape(1, E)
    pos_d = pos_q[0, :dl]
    pos_qq = pos_q[0, :ql]

    def run_encoder(x0, kb, pos, bs):
        x = x0
        for l in range(nlayers):
            bf16 = jnp.bfloat16
            x = _enc_layer(
                x, kb, pos, lng, lnb,
                in_proj_w[l].T.astype(bf16), in_proj_b[l].reshape(1, 3 * E),
                out_w[l].T.astype(bf16), out_b[l].reshape(1, E),
                ff1_w[l].T.astype(bf16), ff1_b[l].reshape(1, FF),
                ff2_w[l].T.astype(bf16), ff2_b[l].reshape(1, E),
                n1_g[l].reshape(1, E), n1_b[l].reshape(1, E),
                n2_g[l].reshape(1, E), n2_b[l].reshape(1, E),
                first=(l == 0), bs=bs)
        return x

    q_enc = run_encoder(query_embeddings.astype(f32), qkb, pos_qq, bs_q)
    d_enc = run_encoder(document_embeddings.astype(f32), dkb, pos_d, 1)

    score3 = _pool(q_enc, d_enc, wq, cmul, bs=bs_p)
    return score3[:, 0, 0]


# speedup vs baseline: 2.0339x; 1.0710x over previous
import functools
import math

import jax
import jax.numpy as jnp
from jax import lax
from jax.experimental import pallas as pl
from jax.experimental.pallas import tpu as pltpu

E = 512
H = 8
HD = E // H
FF = 2048
KK = 11
_MU = (1.0, 0.9, 0.7, 0.5, 0.3, 0.1, -0.1, -0.3, -0.5, -0.7, -0.9)
_SIG = (0.001,) + (0.1,) * 10
_SCALE = 1.0 / math.sqrt(HD)
_NEG = -1e9
_VMEM = pltpu.CompilerParams(
    dimension_semantics=("parallel",),
    vmem_limit_bytes=52 * 1024 * 1024,
)


def _ln2d(x, g, b):
    m = jnp.mean(x, axis=-1, keepdims=True)
    v = jnp.mean((x - m) ** 2, axis=-1, keepdims=True)
    return (x - m) / jnp.sqrt(v + 1e-5) * g + b


def _bf(v):
    return v.astype(jnp.bfloat16)


def _dot(a, b_, dims):
    return lax.dot_general(a, b_, (dims, ((), ())),
                           preferred_element_type=jnp.float32)


def _layer_compute(x, kb, iwT_ref, ib_ref, owT_ref, ob_ref,
                   f1T_ref, f1b_ref, f2T_ref, f2b_ref,
                   g1_ref, b1_ref, g2_ref, b2_ref):
    qkv = _dot(_bf(x), iwT_ref[...], ((1,), (0,))) + ib_ref[...]
    ctxs = []
    for h in range(H):
        qh = _bf(qkv[:, h * HD:(h + 1) * HD] * _SCALE)
        kh = _bf(qkv[:, E + h * HD:E + (h + 1) * HD])
        vh = _bf(qkv[:, 2 * E + h * HD:2 * E + (h + 1) * HD])
        sc = _dot(qh, kh, ((1,), (1,))) + kb
        mx = jnp.max(sc, axis=-1, keepdims=True)
        ex = jnp.exp(sc - mx)
        a = ex / jnp.sum(ex, axis=-1, keepdims=True)
        ctxs.append(_dot(_bf(a), vh, ((1,), (0,))))
    ctx = jnp.concatenate(ctxs, axis=-1)
    att = _dot(_bf(ctx), owT_ref[...], ((1,), (0,))) + ob_ref[...]
    x1 = _ln2d(x + att, g1_ref[...], b1_ref[...])
    hff = jnp.maximum(
        _dot(_bf(x1), f1T_ref[...], ((1,), (0,))) + f1b_ref[...], 0.0)
    ff = _dot(_bf(hff), f2T_ref[...], ((1,), (0,))) + f2b_ref[...]
    return _ln2d(x1 + ff, g2_ref[...], b2_ref[...])


def _enc_body(x_ref, kb_ref, pos_ref, lng_ref, lnb_ref, *rest, bs, s, nlayers):
    wrefs = rest[:-1]
    o_ref = rest[-1]
    ss = bs * s
    x = (x_ref[...] + pos_ref[...][None, :, :]).reshape(ss, E)
    x = _ln2d(x, lng_ref[...], lnb_ref[...])
    kb = kb_ref[...].reshape(1, ss)
    if bs > 1:
        r = lax.broadcasted_iota(jnp.int32, (ss, ss), 0) // s
        c = lax.broadcasted_iota(jnp.int32, (ss, ss), 1) // s
        kb = kb + jnp.where(r == c, 0.0, 2.0 * _NEG)
    for l in range(nlayers):
        x = _layer_compute(x, kb, *wrefs[l * 12:(l + 1) * 12])
    o_ref[...] = x.reshape(bs, s, E)


def _encoder(x, kb, pos, lng, lnb, wflat, *, bs, nlayers):
    b, s, _ = x.shape
    full = lambda a: pl.BlockSpec(a.shape, lambda i: (0,) * a.ndim)
    body = functools.partial(_enc_body, bs=bs, s=s, nlayers=nlayers)
    return pl.pallas_call(
        body,
        out_shape=jax.ShapeDtypeStruct((b, s, E), jnp.float32),
        grid=(b // bs,),
        in_specs=[
            pl.BlockSpec((bs, s, E), lambda i: (i, 0, 0)),
            pl.BlockSpec((1, 1, bs * s), lambda i: (i, 0, 0)),
            full(pos), full(lng), full(lnb),
        ] + [full(w) for w in wflat],
        out_specs=pl.BlockSpec((bs, s, E), lambda i: (i, 0, 0)),
        compiler_params=_VMEM,
        name=f"encoder_s{s}",
    )(x, kb, pos, lng, lnb, *wflat)


def _pool_body(q_ref, d_ref, w_ref, cm_ref, o_ref, *, bs, nq, nd):
    q = q_ref[...].reshape(bs * nq, E)
    d = d_ref[...].reshape(bs * nd, E)
    qn = q / (jnp.sqrt(jnp.sum(q * q, axis=-1, keepdims=True)) + 1e-13)
    dn = d / (jnp.sqrt(jnp.sum(d * d, axis=-1, keepdims=True)) + 1e-13)
    cos = _dot(_bf(qn), _bf(dn), ((1,), (1,)))
    cm = cm_ref[...].reshape(1, bs * nd)
    wq = w_ref[...].reshape(bs * nq, KK)
    scores = []
    for i in range(bs):
        ci = cos[i * nq:(i + 1) * nq, i * nd:(i + 1) * nd]
        cmi = cm[:, i * nd:(i + 1) * nd]
        cols = []
        for k in range(KK):
            inv = 1.0 / (2.0 * _SIG[k] ** 2)
            ex = jnp.exp((ci - _MU[k]) ** 2 * (-inv)) * cmi
            cols.append(jnp.sum(ex, axis=-1, keepdims=True))
        pkq = jnp.concatenate(cols, axis=-1)
        lg = jnp.log(jnp.maximum(pkq, 1e-10))
        scores.append(jnp.sum(lg * wq[i * nq:(i + 1) * nq, :]))
    o_ref[...] = jnp.stack(scores).reshape(bs, 1, 1)


def _pool(q, d, w, cm, *, bs):
    b, nq, _ = q.shape
    nd = d.shape[1]
    body = functools.partial(_pool_body, bs=bs, nq=nq, nd=nd)
    return pl.pallas_call(
        body,
        out_shape=jax.ShapeDtypeStruct((b, 1, 1), jnp.float32),
        grid=(b // bs,),
        in_specs=[
            pl.BlockSpec((bs, nq, E), lambda i: (i, 0, 0)),
            pl.BlockSpec((bs, nd, E), lambda i: (i, 0, 0)),
            pl.BlockSpec((bs, nq, KK), lambda i: (i, 0, 0)),
            pl.BlockSpec((1, 1, bs * nd), lambda i: (i, 0, 0)),
        ],
        out_specs=pl.BlockSpec((bs, 1, 1), lambda i: (i, 0, 0)),
        compiler_params=_VMEM,
        name="rbf_pool",
    )(q, d, w, cm)


def kernel(query_embeddings, document_embeddings, query_pad_oov_mask,
           document_pad_oov_mask, pos_q, ln_g, ln_b, in_proj_w, in_proj_b,
           out_w, out_b, ff1_w, ff1_b, ff2_w, ff2_b, n1_g, n1_b, n2_g, n2_b,
           dense_w, pos_bias, pos_bias_abs):
    f32 = jnp.float32
    b, ql, _ = query_embeddings.shape
    dl = document_embeddings.shape[1]
    nlayers = in_proj_w.shape[0]
    bs_q = 8 if b % 8 == 0 else 1
    bs_p = 4 if b % 4 == 0 else 1

    qmask = query_pad_oov_mask.astype(f32)
    dmask = document_pad_oov_mask.astype(f32)

    qkb = jnp.where(qmask > 0, 0.0, _NEG).reshape(b // bs_q, 1, bs_q * ql)
    dkb = jnp.where(dmask > 0, 0.0, _NEG).reshape(b, 1, dl)

    nbins = pos_bias.shape[0] - 1
    bin_pct = 1.0 / nbins
    doc_len = dmask.sum(1)
    pos_range = (jnp.round(doc_len * bin_pct)[:, None]
                 * jnp.arange(nbins, dtype=f32)).astype(jnp.int32)
    marks = jnp.zeros_like(dmask).at[
        jnp.arange(b)[:, None], pos_range].set(1.0)
    pos_idx = (jnp.cumsum(marks, axis=1) * dmask).astype(jnp.int32)
    pbm = pos_bias[pos_idx]
    abs_factors = pos_bias_abs.shape[0] - 1
    abs_steps = math.ceil(pos_q.shape[1] / abs_factors)
    pos_selects = jnp.repeat(
        jnp.arange(1, abs_factors + 1), abs_steps).astype(f32)[:dl]
    abs_idx = (pos_selects[None, :] * dmask).astype(jnp.int32)
    abm = pos_bias_abs[abs_idx]
    cmul = (dmask * pbm * abm).reshape(b // bs_p, 1, bs_p * dl)
    wq = (qmask[:, :, None] * dense_w[0][None, None, :]).astype(f32)

    lng = ln_g.reshape(1, E)
    lnb = ln_b.reshape(1, E)
    pos_d = pos_q[0, :dl]
    pos_qq = pos_q[0, :ql]

    bf16 = jnp.bfloat16
    wflat = []
    for l in range(nlayers):
        wflat += [
            in_proj_w[l].T.astype(bf16), in_proj_b[l].reshape(1, 3 * E),
            out_w[l].T.astype(bf16), out_b[l].reshape(1, E),
            ff1_w[l].T.astype(bf16), ff1_b[l].reshape(1, FF),
            ff2_w[l].T.astype(bf16), ff2_b[l].reshape(1, E),
            n1_g[l].reshape(1, E), n1_b[l].reshape(1, E),
            n2_g[l].reshape(1, E), n2_b[l].reshape(1, E),
        ]

    def run_encoder(x0, kb, pos, bs):
        return _encoder(x0, kb, pos, lng, lnb, wflat, bs=bs, nlayers=nlayers)

    q_enc = run_encoder(query_embeddings.astype(f32), qkb, pos_qq, bs_q)
    d_enc = run_encoder(document_embeddings.astype(f32), dkb, pos_d, 1)

    score3 = _pool(q_enc, d_enc, wq, cmul, bs=bs_p)
    return score3[:, 0, 0]
